# baseline (speedup 1.0000x reference)
import math

import jax
import jax.numpy as jnp
from jax import lax
from jax.experimental import pallas as pl
from jax.experimental.pallas import tpu as pltpu

_S = 64.0
_COS_M = math.cos(0.5)
_SIN_M = math.sin(0.5)


def _arcface_body(lab_ref, cos_ref, out_ref):
    c = cos_ref[...]
    bb, bc = c.shape
    cols = lax.broadcasted_iota(jnp.int32, (bb, bc), 1)
    mask = cols == lab_ref[...]
    ci = jnp.sum(jnp.where(mask, c, 0.0), axis=1, keepdims=True)
    fix = _S * (ci * _COS_M - jnp.sqrt(jnp.maximum(1.0 - ci * ci, 0.0)) * _SIN_M)
    out_ref[...] = jnp.where(mask, fix, _S * c)


def kernel(cosine, label):
    B, C = cosine.shape
    bb = 16
    return pl.pallas_call(
        _arcface_body,
        grid=(B // bb,),
        in_specs=[
            pl.BlockSpec((bb, 1), lambda i: (i, 0)),
            pl.BlockSpec((bb, C), lambda i: (i, 0)),
        ],
        out_specs=pl.BlockSpec((bb, C), lambda i: (i, 0)),
        out_shape=jax.ShapeDtypeStruct((B, C), cosine.dtype),
        compiler_params=pltpu.CompilerParams(dimension_semantics=("parallel",)),
    )(label.reshape(B, 1), cosine)
